# baseline (speedup 1.0000x reference)
import functools

import jax
import jax.numpy as jnp
from jax import lax
from jax.experimental import pallas as pl
from jax.experimental.pallas import tpu as pltpu
from jax.experimental.pallas import tpu_sc as plsc

B = 16384
D = 32
L = 16
RW = 128
PACK = RW // D

_info = plsc.get_sparse_core_info()
NC = _info.num_cores
NS = _info.num_subcores
NW = NC * NS
BPW = B // NW
HB = 256
CHUNK = 128
NHALF = BPW // HB

_mesh = plsc.VectorSubcoreMesh(core_axis_name="c", subcore_axis_name="s")


@functools.partial(
    pl.kernel,
    mesh=_mesh,
    out_type=jax.ShapeDtypeStruct((B,), jnp.float32),
    compiler_params=pltpu.CompilerParams(needs_layout_passes=False),
    scratch_types=[
        pltpu.VMEM((BPW,), jnp.int32),
        pltpu.VMEM((BPW,), jnp.int32),
        pltpu.VMEM((BPW,), jnp.int32),
        pltpu.VMEM((BPW,), jnp.int32),
        pltpu.VMEM((HB, RW), jnp.float32),
        pltpu.VMEM((HB, RW), jnp.float32),
        pltpu.VMEM((BPW,), jnp.float32),
        pltpu.SemaphoreType.DMA,
    ],
)
def _svd_dot(uidx_hbm, iidx_hbm, ufac_hbm, ifac_hbm, out_hbm,
             uidx_v, iidx_v, urow_v, irow_v, ubuf, ibuf, out_v, sem):
    wid = lax.axis_index("s") * NC + lax.axis_index("c")
    base = wid * BPW

    pltpu.sync_copy(uidx_hbm.at[pl.ds(base, BPW)], uidx_v)
    pltpu.sync_copy(iidx_hbm.at[pl.ds(base, BPW)], iidx_v)

    def shift_body(g, carry):
        sl = pl.ds(g * L, L)
        urow_v[sl] = lax.shift_right_logical(uidx_v[sl], 2)
        irow_v[sl] = lax.shift_right_logical(iidx_v[sl], 2)
        return carry

    lax.fori_loop(0, BPW // L, shift_body, jnp.int32(0))

    for h in range(NHALF):
        copies = []
        for c in range(HB // CHUNK):
            o = h * HB + c * CHUNK
            copies.append(pltpu.async_copy(
                ufac_hbm.at[urow_v.at[pl.ds(o, CHUNK)]],
                ubuf.at[pl.ds(c * CHUNK, CHUNK)], sem))
            copies.append(pltpu.async_copy(
                ifac_hbm.at[irow_v.at[pl.ds(o, CHUNK)]],
                ibuf.at[pl.ds(c * CHUNK, CHUNK)], sem))
        for cp in copies:
            cp.wait()

        def group_body(g, carry):
            rows = g * L + lax.iota(jnp.int32, L)
            sl = pl.ds(h * HB + g * L, L)
            ucol = (uidx_v[sl] & PACK - 1) * D
            icol = (iidx_v[sl] & PACK - 1) * D
            acc = jnp.zeros((L,), jnp.float32)
            for d in range(D):
                u = plsc.load_gather(ubuf, [rows, ucol + d])
                v = plsc.load_gather(ibuf, [rows, icol + d])
                acc = acc + u * v
            out_v[sl] = acc
            return carry

        lax.fori_loop(0, HB // L, group_body, jnp.int32(0))

    pltpu.sync_copy(out_v, out_hbm.at[pl.ds(base, BPW)])


def kernel(user_idx, item_idx, user_factors, item_factors):
    ufac = user_factors.reshape(-1, RW)
    ifac = item_factors.reshape(-1, RW)
    return _svd_dot(user_idx, item_idx, ufac, ifac)


# speedup vs baseline: 4.1279x; 4.1279x over previous
import functools

import jax
import jax.numpy as jnp
from jax import lax
from jax.experimental import pallas as pl
from jax.experimental.pallas import tpu as pltpu
from jax.experimental.pallas import tpu_sc as plsc

B = 16384
D = 32
L = 16
NSUB = 8
NGRP = D // NSUB
TW = 128

_info = plsc.get_sparse_core_info()
NC = _info.num_cores
NS = _info.num_subcores
NW = NC * NS
BPW = B // NW
WAVE = 4
NSLOT = 2 * WAVE

_mesh = plsc.VectorSubcoreMesh(core_axis_name="c", subcore_axis_name="s")

_IOTA = lambda: lax.iota(jnp.int32, L)


@functools.partial(
    pl.kernel,
    mesh=_mesh,
    out_type=jax.ShapeDtypeStruct((B,), jnp.float32),
    compiler_params=pltpu.CompilerParams(needs_layout_passes=False),
    scratch_types=[
        pltpu.VMEM((BPW,), jnp.int32),
        pltpu.VMEM((BPW,), jnp.int32),
        pltpu.VMEM((NSLOT, NGRP, NSUB, TW), jnp.float32),
        pltpu.VMEM((NSLOT, NGRP, NSUB, TW), jnp.float32),
        pltpu.VMEM((L, L), jnp.float32),
        pltpu.VMEM((BPW,), jnp.float32),
        pltpu.SemaphoreType.DMA,
    ],
)
def _svd_dot(uidx_hbm, iidx_hbm, ufac_hbm, ifac_hbm, out_hbm,
             uidx_v, iidx_v, ublk, iblk, pbuf, out_v, sem):
    wid = lax.axis_index("s") * NC + lax.axis_index("c")
    base = wid * BPW

    pltpu.sync_copy(uidx_hbm.at[pl.ds(base, BPW)], uidx_v)
    pltpu.sync_copy(iidx_hbm.at[pl.ds(base, BPW)], iidx_v)

    iota = _IOTA()
    a_lo = lax.shift_right_logical(iota, 3)
    a_hi = a_lo + 2
    s_all = iota & 7

    def fire_wave(u16, i16, k0, slot0):
        copies = []
        for k in range(WAVE):
            cu = pl.multiple_of(
                lax.shift_right_logical(u16[k0 + k], 7) * TW, TW)
            ci = pl.multiple_of(
                lax.shift_right_logical(i16[k0 + k], 7) * TW, TW)
            copies.append(pltpu.async_copy(
                ufac_hbm.at[:, :, pl.ds(cu, TW)], ublk.at[slot0 + k], sem))
            copies.append(pltpu.async_copy(
                ifac_hbm.at[:, :, pl.ds(ci, TW)], iblk.at[slot0 + k], sem))
        return copies

    def extract_wave(u16, i16, k0, slot0):
        for k in range(WAVE):
            lu = jnp.full((L,), u16[k0 + k] & (TW - 1), jnp.int32)
            li = jnp.full((L,), i16[k0 + k] & (TW - 1), jnp.int32)
            slot = jnp.full((L,), slot0 + k, jnp.int32)
            u_lo = plsc.load_gather(ublk, [slot, a_lo, s_all, lu])
            u_hi = plsc.load_gather(ublk, [slot, a_hi, s_all, lu])
            v_lo = plsc.load_gather(iblk, [slot, a_lo, s_all, li])
            v_hi = plsc.load_gather(iblk, [slot, a_hi, s_all, li])
            pbuf[k0 + k, :] = u_lo * v_lo + u_hi * v_hi

    def group_body(g, carry):
        sl = pl.ds(g * L, L)
        u16 = uidx_v[sl]
        i16 = iidx_v[sl]
        prev = fire_wave(u16, i16, 0, 0)
        for w in range(1, L // WAVE + 1):
            if w <= L // WAVE - 1:
                nxt = fire_wave(u16, i16, w * WAVE, (w % 2) * WAVE)
            else:
                nxt = None
            for cp in prev:
                cp.wait()
            extract_wave(u16, i16, (w - 1) * WAVE, ((w - 1) % 2) * WAVE)
            prev = nxt
        acc = jnp.zeros((L,), jnp.float32)
        for l in range(L):
            acc = acc + plsc.load_gather(pbuf, [iota, jnp.full((L,), l, jnp.int32)])
        out_v[sl] = acc
        return carry

    lax.fori_loop(0, BPW // L, group_body, jnp.int32(0))

    pltpu.sync_copy(out_v, out_hbm.at[pl.ds(base, BPW)])


def kernel(user_idx, item_idx, user_factors, item_factors):
    ufac = user_factors.T.reshape(NGRP, NSUB, -1)
    ifac = item_factors.T.reshape(NGRP, NSUB, -1)
    return _svd_dot(user_idx, item_idx, ufac, ifac)


# speedup vs baseline: 4.1359x; 1.0019x over previous
import functools

import jax
import jax.numpy as jnp
from jax import lax
from jax.experimental import pallas as pl
from jax.experimental.pallas import tpu as pltpu
from jax.experimental.pallas import tpu_sc as plsc

B = 16384
D = 32
L = 16
NSUB = 8
NGRP = D // NSUB
TW = 128

_info = plsc.get_sparse_core_info()
NC = _info.num_cores
NS = _info.num_subcores
NW = NC * NS
BPW = B // NW
WAVE = 4
NSLOT = 2 * WAVE

_mesh = plsc.VectorSubcoreMesh(core_axis_name="c", subcore_axis_name="s")

_IOTA = lambda: lax.iota(jnp.int32, L)


@functools.partial(
    pl.kernel,
    mesh=_mesh,
    out_type=jax.ShapeDtypeStruct((B,), jnp.float32),
    compiler_params=pltpu.CompilerParams(needs_layout_passes=False),
    scratch_types=[
        pltpu.VMEM((BPW,), jnp.int32),
        pltpu.VMEM((BPW,), jnp.int32),
        pltpu.VMEM((NSLOT, NGRP, NSUB, TW), jnp.float32),
        pltpu.VMEM((NSLOT, NGRP, NSUB, TW), jnp.float32),
        pltpu.VMEM((L, L), jnp.float32),
        pltpu.VMEM((BPW,), jnp.float32),
        pltpu.SemaphoreType.DMA,
    ],
)
def _svd_dot(uidx_hbm, iidx_hbm, ufac_hbm, ifac_hbm, out_hbm,
             uidx_v, iidx_v, ublk, iblk, pbuf, out_v, sem):
    wid = lax.axis_index("s") * NC + lax.axis_index("c")
    base = wid * BPW

    pltpu.sync_copy(uidx_hbm.at[pl.ds(base, BPW)], uidx_v)
    pltpu.sync_copy(iidx_hbm.at[pl.ds(base, BPW)], iidx_v)

    iota = _IOTA()
    a_lo = lax.shift_right_logical(iota, 3)
    a_hi = a_lo + 2
    s_all = iota & 7

    def fire_wave(u16, i16, k0, slot0):
        copies = []
        for k in range(WAVE):
            cu = pl.multiple_of(
                lax.shift_right_logical(u16[k0 + k], 7) * TW, TW)
            ci = pl.multiple_of(
                lax.shift_right_logical(i16[k0 + k], 7) * TW, TW)
            for h in range(2):
                ah = pl.ds(2 * h, 2)
                copies.append(pltpu.async_copy(
                    ufac_hbm.at[ah, :, pl.ds(cu, TW)],
                    ublk.at[slot0 + k].at[ah], sem))
                copies.append(pltpu.async_copy(
                    ifac_hbm.at[ah, :, pl.ds(ci, TW)],
                    iblk.at[slot0 + k].at[ah], sem))
        return copies

    def extract_wave(u16, i16, k0, slot0):
        for k in range(WAVE):
            lu = jnp.full((L,), u16[k0 + k] & (TW - 1), jnp.int32)
            li = jnp.full((L,), i16[k0 + k] & (TW - 1), jnp.int32)
            slot = jnp.full((L,), slot0 + k, jnp.int32)
            u_lo = plsc.load_gather(ublk, [slot, a_lo, s_all, lu])
            u_hi = plsc.load_gather(ublk, [slot, a_hi, s_all, lu])
            v_lo = plsc.load_gather(iblk, [slot, a_lo, s_all, li])
            v_hi = plsc.load_gather(iblk, [slot, a_hi, s_all, li])
            pbuf[k0 + k, :] = u_lo * v_lo + u_hi * v_hi

    def group_body(g, carry):
        sl = pl.ds(g * L, L)
        u16 = uidx_v[sl]
        i16 = iidx_v[sl]
        prev = fire_wave(u16, i16, 0, 0)
        for w in range(1, L // WAVE + 1):
            if w <= L // WAVE - 1:
                nxt = fire_wave(u16, i16, w * WAVE, (w % 2) * WAVE)
            else:
                nxt = None
            for cp in prev:
                cp.wait()
            extract_wave(u16, i16, (w - 1) * WAVE, ((w - 1) % 2) * WAVE)
            prev = nxt
        acc = jnp.zeros((L,), jnp.float32)
        for l in range(L):
            acc = acc + plsc.load_gather(pbuf, [iota, jnp.full((L,), l, jnp.int32)])
        out_v[sl] = acc
        return carry

    lax.fori_loop(0, BPW // L, group_body, jnp.int32(0))

    pltpu.sync_copy(out_v, out_hbm.at[pl.ds(base, BPW)])


def kernel(user_idx, item_idx, user_factors, item_factors):
    ufac = user_factors.T.reshape(NGRP, NSUB, -1)
    ifac = item_factors.T.reshape(NGRP, NSUB, -1)
    return _svd_dot(user_idx, item_idx, ufac, ifac)
